# baseline (speedup 1.0000x reference)
import dataclasses
import functools

import jax
import jax.numpy as jnp
from jax import lax
from jax.experimental import pallas as pl
from jax.experimental.pallas import tpu as pltpu
from jax.experimental.pallas import tpu_sc as plsc

_W = 64
_WB = 4
_NL = 16
_NBUF = 4
_NOUT = 2


@functools.partial(jax.jit, static_argnums=(4, 5))
def _sc_embed(token_table, comb3, seq_re, lab_re, bsz, seqlen):
    d = token_table.shape[1]
    nseg = comb3.shape[0]
    mesh = plsc.VectorSubcoreMesh(core_axis_name="core",
                                  subcore_axis_name="subcore")
    n_workers = 32
    n_per = bsz * _NL
    nw = n_per // _W

    cp = pltpu.CompilerParams()
    if "needs_layout_passes" in pltpu.CompilerParams.__dataclass_fields__:
        cp = dataclasses.replace(cp, needs_layout_passes=False)

    @functools.partial(
        pl.kernel,
        out_type=jax.ShapeDtypeStruct((bsz, seqlen, d), jnp.float32),
        mesh=mesh,
        compiler_params=cp,
        scratch_types=[
            pltpu.VMEM((n_per,), jnp.int32),
            pltpu.VMEM((n_per,), jnp.int32),
            pltpu.VMEM((nseg, _NL, d), jnp.float32),
            pltpu.VMEM((_NBUF, _W, d), jnp.float32),
            pltpu.VMEM((_NOUT, _WB, _NL, d), jnp.float32),
            pltpu.SemaphoreType.DMA((_NBUF,)),
            pltpu.SemaphoreType.DMA((_NOUT,)),
            pltpu.SemaphoreType.DMA,
        ],
    )
    def k(tok_hbm, comb_hbm, seq_hbm, lab_hbm, o_hbm,
          iseq_v, lab_v, comb_v, t_v, o_v, gt_sem, w_sem, ld_sem):
        wid = lax.axis_index("subcore") * 2 + lax.axis_index("core")
        base = wid * n_per
        l0 = wid * _NL

        pltpu.async_copy(seq_hbm.at[pl.ds(base, n_per)], iseq_v, ld_sem).wait()
        pltpu.async_copy(lab_hbm.at[pl.ds(base, n_per)], lab_v, ld_sem).wait()
        for s in range(nseg):
            pltpu.async_copy(comb_hbm.at[s].at[pl.ds(l0, _NL)], comb_v.at[s],
                             ld_sem).wait()

        def start_gather(g, b):
            rows = pl.ds(g * _W, _W)
            pltpu.async_copy(tok_hbm.at[iseq_v.at[rows]], t_v.at[b],
                             gt_sem.at[b])

        def wait_gather(b):
            pltpu.make_async_copy(tok_hbm.at[pl.ds(0, _W)], t_v.at[b],
                                  gt_sem.at[b]).wait()

        def wait_write(ob):
            pltpu.make_async_copy(o_v.at[ob],
                                  o_hbm.at[pl.ds(0, _WB), pl.ds(0, _NL)],
                                  w_sem.at[ob]).wait()

        for b in range(_NBUF):
            start_gather(b, b)

        iota16 = lax.broadcasted_iota(jnp.int32, (16,), 0)

        @pl.loop(0, nw // _NBUF)
        def _outer(i):
            for b in range(_NBUF):
                g = i * _NBUF + b
                ob = b % _NOUT
                wait_gather(b)

                @pl.when(g >= _NOUT)
                def _():
                    wait_write(ob)

                @pl.loop(0, _WB)
                def _rows(jb):
                    row0 = g * _W + jb * _NL
                    for jl in range(_NL):
                        lab16 = plsc.load_gather(
                            lab_v, [jnp.full((16,), row0 + jl, jnp.int32)])
                        jl16 = jnp.full((16,), jl, jnp.int32)
                        r = jb * _NL + jl
                        for c in range(0, d, 16):
                            add16 = plsc.load_gather(
                                comb_v, [lab16, jl16, iota16 + c])
                            o_v[ob, jb, jl, pl.ds(c, 16)] = (
                                t_v[b, r, pl.ds(c, 16)] + add16)

                pltpu.async_copy(
                    o_v.at[ob],
                    o_hbm.at[pl.ds(g * _WB, _WB), pl.ds(l0, _NL)],
                    w_sem.at[ob])

                @pl.when(g + _NBUF < nw)
                def _():
                    start_gather(g + _NBUF, b)

        for ob in range(_NOUT):
            wait_write(ob)

    return k(token_table, comb3, seq_re, lab_re)


def kernel(sequence, segment_label, token_table, segment_table, pe):
    bsz, seqlen = sequence.shape
    d = token_table.shape[1]
    nsl = seqlen // _NL
    seq_re = sequence.reshape(bsz, nsl, _NL).transpose(1, 0, 2).reshape(-1)
    lab_re = segment_label.reshape(bsz, nsl, _NL).transpose(1, 0, 2).reshape(-1)
    comb3 = segment_table[:, None, :] + pe[0][None, :, :]
    return _sc_embed(token_table, comb3, seq_re, lab_re, bsz, seqlen)


# speedup vs baseline: 3.0830x; 3.0830x over previous
import dataclasses
import functools

import jax
import jax.numpy as jnp
from jax import lax
from jax.experimental import pallas as pl
from jax.experimental.pallas import tpu as pltpu
from jax.experimental.pallas import tpu_sc as plsc

_W = 64
_WB = 4
_NL = 16
_NBUF = 4
_NOUT = 2


@functools.partial(jax.jit, static_argnums=(4, 5))
def _sc_embed(token_table, comb3, seq_re, lab_re, bsz, seqlen):
    d = token_table.shape[1]
    nseg = comb3.shape[0]
    mesh = plsc.VectorSubcoreMesh(core_axis_name="core",
                                  subcore_axis_name="subcore")
    n_workers = 32
    n_per = bsz * _NL
    nw = n_per // _W

    cp = pltpu.CompilerParams()
    if "needs_layout_passes" in pltpu.CompilerParams.__dataclass_fields__:
        cp = dataclasses.replace(cp, needs_layout_passes=False)

    @functools.partial(
        pl.kernel,
        out_type=jax.ShapeDtypeStruct((bsz, seqlen, d), jnp.float32),
        mesh=mesh,
        compiler_params=cp,
        scratch_types=[
            pltpu.VMEM((n_per,), jnp.int32),
            pltpu.VMEM((n_per,), jnp.int32),
            pltpu.VMEM((nseg, _NL, d), jnp.float32),
            pltpu.VMEM((_NBUF, _W, d), jnp.float32),
            pltpu.VMEM((_NOUT, _WB, _NL, d), jnp.float32),
            pltpu.SemaphoreType.DMA((_NBUF,)),
            pltpu.SemaphoreType.DMA((_NOUT,)),
            pltpu.SemaphoreType.DMA,
        ],
    )
    def k(tok_hbm, comb_hbm, seq_hbm, lab_hbm, o_hbm,
          iseq_v, lab_v, comb_v, t_v, o_v, gt_sem, w_sem, ld_sem):
        wid = lax.axis_index("subcore") * 2 + lax.axis_index("core")
        base = wid * n_per
        l0 = wid * _NL

        pltpu.async_copy(seq_hbm.at[pl.ds(base, n_per)], iseq_v, ld_sem).wait()
        pltpu.async_copy(lab_hbm.at[pl.ds(base, n_per)], lab_v, ld_sem).wait()
        for s in range(nseg):
            pltpu.async_copy(comb_hbm.at[s].at[pl.ds(l0, _NL)], comb_v.at[s],
                             ld_sem).wait()

        def start_gather(g, b):
            rows = pl.ds(g * _W, _W)
            pltpu.async_copy(tok_hbm.at[iseq_v.at[rows]], t_v.at[b],
                             gt_sem.at[b])

        def wait_gather(b):
            pltpu.make_async_copy(tok_hbm.at[pl.ds(0, _W)], t_v.at[b],
                                  gt_sem.at[b]).wait()

        def wait_write(ob):
            pltpu.make_async_copy(o_v.at[ob],
                                  o_hbm.at[pl.ds(0, _WB), pl.ds(0, _NL)],
                                  w_sem.at[ob]).wait()

        for b in range(_NBUF):
            start_gather(b, b)

        iota16 = lax.broadcasted_iota(jnp.int32, (16,), 0)

        @pl.loop(0, nw // _NBUF)
        def _outer(i):
            for b in range(_NBUF):
                g = i * _NBUF + b
                ob = b % _NOUT
                wait_gather(b)

                @pl.when(g >= _NOUT)
                def _():
                    wait_write(ob)

                @pl.loop(0, _WB)
                def _rows(jb):
                    row0 = g * _W + jb * _NL
                    labs16 = lab_v[pl.ds(row0, _NL)]
                    for jl in range(_NL):
                        lab_s = labs16[jl]
                        r = jb * _NL + jl
                        for c0 in range(0, d, 64):
                            ts = [t_v[b, r, pl.ds(c0 + 16 * kk, 16)]
                                  for kk in range(4)]
                            cs = [comb_v[lab_s, jl, pl.ds(c0 + 16 * kk, 16)]
                                  for kk in range(4)]
                            for kk in range(4):
                                o_v[ob, jb, jl, pl.ds(c0 + 16 * kk, 16)] = (
                                    ts[kk] + cs[kk])

                pltpu.async_copy(
                    o_v.at[ob],
                    o_hbm.at[pl.ds(g * _WB, _WB), pl.ds(l0, _NL)],
                    w_sem.at[ob])

                @pl.when(g + _NBUF < nw)
                def _():
                    start_gather(g + _NBUF, b)

        for ob in range(_NOUT):
            wait_write(ob)

    return k(token_table, comb3, seq_re, lab_re)


def kernel(sequence, segment_label, token_table, segment_table, pe):
    bsz, seqlen = sequence.shape
    d = token_table.shape[1]
    nsl = seqlen // _NL
    seq_re = sequence.reshape(bsz, nsl, _NL).transpose(1, 0, 2).reshape(-1)
    lab_re = segment_label.reshape(bsz, nsl, _NL).transpose(1, 0, 2).reshape(-1)
    comb3 = segment_table[:, None, :] + pe[0][None, :, :]
    return _sc_embed(token_table, comb3, seq_re, lab_re, bsz, seqlen)
